# Initial kernel scaffold; baseline (speedup 1.0000x reference)
import jax
import jax.numpy as jnp
from jax.experimental import pallas as pl


def kernel(x, edge_index, W1, b1, W2, b2):
    raise NotImplementedError("write your pallas kernel here")


# speedup vs baseline: 7.8561x; 7.8561x over previous
import functools

import jax
import jax.numpy as jnp
from jax import lax
from jax.experimental import pallas as pl
from jax.experimental.pallas import tpu as pltpu
from jax.experimental.pallas import tpu_sc as plsc

_F32 = jnp.float32
_I32 = jnp.int32

_NC = 2
_NS = 16
_NW = _NC * _NS
_CH = 128
_ROWBLK = 512


def _agg1_body(y1, srcl, dstl, z32, z16, ones_h, s1, l16,
               acc32, acc16, src_v, dst_v, midx_v, rows_v, ones_v,
               *, chunks_per_tile, npad):
    c = lax.axis_index("c")
    s = lax.axis_index("s")
    wid = c * _NS + s
    rpt32 = npad // _NS
    rpt16 = 2 * npad // _NS
    pltpu.sync_copy(z32.at[pl.ds(0, rpt32)], acc32.at[pl.ds(s * rpt32, rpt32)])
    pltpu.sync_copy(z16, acc16.at[pl.ds(s * rpt16, rpt16)])
    pltpu.sync_copy(ones_h, ones_v)
    plsc.subcore_barrier()
    base = wid * chunks_per_tile

    @pl.loop(0, chunks_per_tile)
    def _chunk(i):
        row = base + i
        pltpu.sync_copy(srcl.at[row], src_v.at[0])
        pltpu.sync_copy(dstl.at[row], dst_v.at[0])
        pltpu.sync_copy(y1.at[src_v.at[0]], rows_v)
        for j in range(_CH // 16):
            sl = pl.ds(j * 16, 16)
            sv = src_v[0, sl]
            dv = dst_v[0, sl]
            midx_v[0, sl] = dv + jnp.where(sv == dv, npad, 0).astype(_I32)
        pltpu.sync_copy(rows_v, acc32.at[dst_v.at[0]], add=True)
        pltpu.sync_copy(ones_v, acc16.at[midx_v.at[0]], add=True)

    plsc.subcore_barrier()
    sl32 = pl.ds(s * rpt32, rpt32)
    sl16 = pl.ds(s * rpt16, rpt16)
    pltpu.sync_copy(acc32.at[sl32], s1.at[c, sl32])
    pltpu.sync_copy(acc16.at[sl16], l16.at[c, sl16])


def _agg2_body(h, srcl, dstl, z32, s2,
               acc32, src_v, dst_v, rows_v,
               *, chunks_per_tile, npad):
    c = lax.axis_index("c")
    s = lax.axis_index("s")
    wid = c * _NS + s
    rpt32 = npad // _NS
    pltpu.sync_copy(z32.at[pl.ds(0, rpt32)], acc32.at[pl.ds(s * rpt32, rpt32)])
    plsc.subcore_barrier()
    base = wid * chunks_per_tile

    @pl.loop(0, chunks_per_tile)
    def _chunk(i):
        row = base + i
        pltpu.sync_copy(srcl.at[row], src_v.at[0])
        pltpu.sync_copy(dstl.at[row], dst_v.at[0])
        pltpu.sync_copy(h.at[src_v.at[0]], rows_v)
        pltpu.sync_copy(rows_v, acc32.at[dst_v.at[0]], add=True)

    plsc.subcore_barrier()
    sl32 = pl.ds(s * rpt32, rpt32)
    pltpu.sync_copy(acc32.at[sl32], s2.at[c, sl32])


def _mm1_body(x_ref, w_ref, o_ref):
    o_ref[...] = jnp.dot(x_ref[...], w_ref[...],
                         preferred_element_type=_F32)


def _mid_body(s1_ref, la_ref, lb_ref, y1_ref, b1_ref, h_ref, st_ref):
    seg = s1_ref[0] + s1_ref[1]
    la = la_ref[0] + la_ref[1]
    lb = lb_ref[0] + lb_ref[1]
    loop_cnt = lb[:, 0:1]
    deg_raw = la[:, 0:1] + loop_cnt
    missing = jnp.where(loop_cnt > 0.0, 0.0, 1.0)
    deg = jnp.maximum(deg_raw + missing, 1.0)
    y1 = y1_ref[...]
    h = jnp.maximum((seg + y1 * missing) / deg + b1_ref[...], 0.0)
    h_ref[...] = h
    nrows = missing.shape[0]
    st_ref[...] = jnp.concatenate(
        [missing, deg, jnp.zeros((nrows, 14), _F32)], axis=1)


def _fin_body(s2_ref, h_ref, st_ref, w2_ref, b2_ref, o_ref):
    seg = s2_ref[0] + s2_ref[1]
    missing = st_ref[:, 0:1]
    deg = st_ref[:, 1:2]
    aggr = (seg + h_ref[...] * missing) / deg
    z = jnp.dot(aggr, w2_ref[...], preferred_element_type=_F32) + b2_ref[...]
    m = jnp.max(z, axis=1, keepdims=True)
    e = jnp.exp(z - m)
    lse = jnp.log(jnp.sum(e, axis=1, keepdims=True))
    o_ref[...] = z - m - lse


def kernel(x, edge_index, W1, b1, W2, b2):
    n, d_in = x.shape
    e = edge_index.shape[1]
    d_hid = W1.shape[1]
    d_out = W2.shape[1]

    npad = ((n + 2 * _ROWBLK - 1) // _ROWBLK) * _ROWBLK
    trash = npad - 1
    chunks_total = -(-e // _CH)
    chunks_per_tile = -(-chunks_total // _NW)
    epad = chunks_per_tile * _NW * _CH
    nblk = npad // _ROWBLK

    src = edge_index[0]
    dst = edge_index[1]
    pad = jnp.full((epad - e,), trash, _I32)
    srcl = jnp.concatenate([src, pad]).reshape(-1, _CH)
    dstl = jnp.concatenate([dst, pad]).reshape(-1, _CH)
    x_pad = jnp.pad(x, ((0, npad - n), (0, 0)))
    z32 = jnp.zeros((npad // _NS + 8, d_hid), _F32)
    z16 = jnp.zeros((2 * npad // _NS, 16), _F32)
    ones_h = jnp.zeros((_CH, 16), _F32).at[:, 0].set(1.0)

    y1p = pl.pallas_call(
        _mm1_body,
        grid=(nblk,),
        in_specs=[pl.BlockSpec((_ROWBLK, d_in), lambda i: (i, 0)),
                  pl.BlockSpec((d_in, d_hid), lambda i: (0, 0))],
        out_specs=pl.BlockSpec((_ROWBLK, d_hid), lambda i: (i, 0)),
        out_shape=jax.ShapeDtypeStruct((npad, d_hid), _F32),
    )(x_pad, W1)

    mesh = plsc.VectorSubcoreMesh(core_axis_name="c", subcore_axis_name="s",
                                  num_cores=_NC, num_subcores=_NS)
    sc_params = pltpu.CompilerParams(use_tc_tiling_on_sc=False)

    agg1 = pl.kernel(
        functools.partial(_agg1_body, chunks_per_tile=chunks_per_tile,
                          npad=npad),
        out_type=[jax.ShapeDtypeStruct((_NC, npad, d_hid), _F32),
                  jax.ShapeDtypeStruct((_NC, 2 * npad, 16), _F32)],
        mesh=mesh,
        scratch_types=[
            pltpu.VMEM_SHARED((npad, d_hid), _F32),
            pltpu.VMEM_SHARED((2 * npad, 16), _F32),
            pltpu.VMEM((1, _CH), _I32),
            pltpu.VMEM((1, _CH), _I32),
            pltpu.VMEM((1, _CH), _I32),
            pltpu.VMEM((_CH, d_hid), _F32),
            pltpu.VMEM((_CH, 16), _F32),
        ],
        compiler_params=sc_params,
    )
    s1, l16 = agg1(y1p, srcl, dstl, z32, z16, ones_h)

    nboff = npad // _ROWBLK
    h, stats = pl.pallas_call(
        _mid_body,
        grid=(nblk,),
        in_specs=[pl.BlockSpec((_NC, _ROWBLK, d_hid), lambda i: (0, i, 0)),
                  pl.BlockSpec((_NC, _ROWBLK, 16), lambda i: (0, i, 0)),
                  pl.BlockSpec((_NC, _ROWBLK, 16),
                               lambda i: (0, i + nboff, 0)),
                  pl.BlockSpec((_ROWBLK, d_hid), lambda i: (i, 0)),
                  pl.BlockSpec((1, d_hid), lambda i: (0, 0))],
        out_specs=[pl.BlockSpec((_ROWBLK, d_hid), lambda i: (i, 0)),
                   pl.BlockSpec((_ROWBLK, 16), lambda i: (i, 0))],
        out_shape=[jax.ShapeDtypeStruct((npad, d_hid), _F32),
                   jax.ShapeDtypeStruct((npad, 16), _F32)],
    )(s1, l16, l16, y1p, b1.reshape(1, d_hid))

    agg2 = pl.kernel(
        functools.partial(_agg2_body, chunks_per_tile=chunks_per_tile,
                          npad=npad),
        out_type=jax.ShapeDtypeStruct((_NC, npad, d_hid), _F32),
        mesh=mesh,
        scratch_types=[
            pltpu.VMEM_SHARED((npad, d_hid), _F32),
            pltpu.VMEM((1, _CH), _I32),
            pltpu.VMEM((1, _CH), _I32),
            pltpu.VMEM((_CH, d_hid), _F32),
        ],
        compiler_params=sc_params,
    )
    s2 = agg2(h, srcl, dstl, z32)

    out = pl.pallas_call(
        _fin_body,
        grid=(nblk,),
        in_specs=[pl.BlockSpec((_NC, _ROWBLK, d_hid), lambda i: (0, i, 0)),
                  pl.BlockSpec((_ROWBLK, d_hid), lambda i: (i, 0)),
                  pl.BlockSpec((_ROWBLK, 16), lambda i: (i, 0)),
                  pl.BlockSpec((d_hid, d_out), lambda i: (0, 0)),
                  pl.BlockSpec((1, d_out), lambda i: (0, 0))],
        out_specs=pl.BlockSpec((_ROWBLK, d_out), lambda i: (i, 0)),
        out_shape=jax.ShapeDtypeStruct((npad, d_out), _F32),
    )(s2, h, stats, W2, b2.reshape(1, d_out))

    return out[:n]


# speedup vs baseline: 9.6455x; 1.2278x over previous
import functools

import jax
import jax.numpy as jnp
from jax import lax
from jax.experimental import pallas as pl
from jax.experimental.pallas import tpu as pltpu
from jax.experimental.pallas import tpu_sc as plsc

_F32 = jnp.float32
_I32 = jnp.int32

_NC = 2
_NS = 16
_NW = _NC * _NS
_CH = 512
_CHR = _CH // 128
_ROWBLK = 512


def _agg1_body(y1, sd, z32, z16, ones_h, s1, l16,
               acc32, acc16, sd_v, midx_v, rows_v, ones_v,
               *, chunks_per_tile, npad):
    c = lax.axis_index("c")
    s = lax.axis_index("s")
    wid = c * _NS + s
    rpt32 = npad // _NS
    rpt16 = 2 * npad // _NS
    pltpu.sync_copy(z32.at[pl.ds(0, rpt32)], acc32.at[pl.ds(s * rpt32, rpt32)])
    pltpu.sync_copy(z16, acc16.at[pl.ds(s * rpt16, rpt16)])
    pltpu.sync_copy(ones_h, ones_v)
    plsc.subcore_barrier()
    base = wid * chunks_per_tile

    @pl.loop(0, chunks_per_tile)
    def _chunk(i):
        row = base + i
        pltpu.sync_copy(sd.at[row], sd_v)
        pltpu.sync_copy(y1.at[sd_v.at[0]], rows_v)
        for j in range(_CH // 16):
            sl = pl.ds(j * 16, 16)
            sv = sd_v[0, sl]
            dv = sd_v[1, sl]
            midx_v[sl] = dv + jnp.where(sv == dv, npad, 0).astype(_I32)
        pltpu.sync_copy(rows_v, acc32.at[sd_v.at[1]], add=True)
        pltpu.sync_copy(ones_v, acc16.at[midx_v], add=True)

    plsc.subcore_barrier()
    sl32 = pl.ds(s * rpt32, rpt32)
    sl16 = pl.ds(s * rpt16, rpt16)
    pltpu.sync_copy(acc32.at[sl32], s1.at[c, sl32])
    pltpu.sync_copy(acc16.at[sl16], l16.at[c, sl16])


def _agg2_body(h, sd, z32, s2,
               acc32, sd_v, rows_v,
               *, chunks_per_tile, npad):
    c = lax.axis_index("c")
    s = lax.axis_index("s")
    wid = c * _NS + s
    rpt32 = npad // _NS
    pltpu.sync_copy(z32.at[pl.ds(0, rpt32)], acc32.at[pl.ds(s * rpt32, rpt32)])
    plsc.subcore_barrier()
    base = wid * chunks_per_tile

    @pl.loop(0, chunks_per_tile)
    def _chunk(i):
        row = base + i
        pltpu.sync_copy(sd.at[row], sd_v)
        pltpu.sync_copy(h.at[sd_v.at[0]], rows_v)
        pltpu.sync_copy(rows_v, acc32.at[sd_v.at[1]], add=True)

    plsc.subcore_barrier()
    sl32 = pl.ds(s * rpt32, rpt32)
    pltpu.sync_copy(acc32.at[sl32], s2.at[c, sl32])


def _mm1_body(x_ref, w_ref, o_ref):
    o_ref[...] = jnp.dot(x_ref[...], w_ref[...],
                         preferred_element_type=_F32)


def _mid_body(s1_ref, la_ref, lb_ref, y1_ref, b1_ref, h_ref, st_ref):
    seg = s1_ref[0] + s1_ref[1]
    la = la_ref[0] + la_ref[1]
    lb = lb_ref[0] + lb_ref[1]
    loop_cnt = lb[:, 0:1]
    deg_raw = la[:, 0:1] + loop_cnt
    missing = jnp.where(loop_cnt > 0.0, 0.0, 1.0)
    deg = jnp.maximum(deg_raw + missing, 1.0)
    y1 = y1_ref[...]
    h = jnp.maximum((seg + y1 * missing) / deg + b1_ref[...], 0.0)
    h_ref[...] = h
    nrows = missing.shape[0]
    st_ref[...] = jnp.concatenate(
        [missing, deg, jnp.zeros((nrows, 14), _F32)], axis=1)


def _fin_body(s2_ref, h_ref, st_ref, w2_ref, b2_ref, o_ref):
    seg = s2_ref[0] + s2_ref[1]
    missing = st_ref[:, 0:1]
    deg = st_ref[:, 1:2]
    aggr = (seg + h_ref[...] * missing) / deg
    z = jnp.dot(aggr, w2_ref[...], preferred_element_type=_F32) + b2_ref[...]
    m = jnp.max(z, axis=1, keepdims=True)
    e = jnp.exp(z - m)
    lse = jnp.log(jnp.sum(e, axis=1, keepdims=True))
    o_ref[...] = z - m - lse


def kernel(x, edge_index, W1, b1, W2, b2):
    n, d_in = x.shape
    e = edge_index.shape[1]
    d_hid = W1.shape[1]
    d_out = W2.shape[1]

    npad = ((n + 2 * _ROWBLK - 1) // _ROWBLK) * _ROWBLK
    trash = npad - 1
    chunks_per_tile = -(-e // (_CH * _NW))
    epad = chunks_per_tile * _NW * _CH
    nblk = npad // _ROWBLK

    src = edge_index[0]
    dst = edge_index[1]
    pad = jnp.full((epad - e,), trash, _I32)
    srcl = jnp.concatenate([src, pad]).reshape(-1, _CH)
    dstl = jnp.concatenate([dst, pad]).reshape(-1, _CH)
    sd = jnp.stack([srcl, dstl], axis=1)
    x_pad = jnp.pad(x, ((0, npad - n), (0, 0)))
    z32 = jnp.zeros((npad // _NS + 8, d_hid), _F32)
    z16 = jnp.zeros((2 * npad // _NS, 16), _F32)
    ones_h = jnp.zeros((_CH, 16), _F32).at[:, 0].set(1.0)

    y1p = pl.pallas_call(
        _mm1_body,
        grid=(nblk,),
        in_specs=[pl.BlockSpec((_ROWBLK, d_in), lambda i: (i, 0)),
                  pl.BlockSpec((d_in, d_hid), lambda i: (0, 0))],
        out_specs=pl.BlockSpec((_ROWBLK, d_hid), lambda i: (i, 0)),
        out_shape=jax.ShapeDtypeStruct((npad, d_hid), _F32),
    )(x_pad, W1)

    mesh = plsc.VectorSubcoreMesh(core_axis_name="c", subcore_axis_name="s",
                                  num_cores=_NC, num_subcores=_NS)
    sc_params = pltpu.CompilerParams(use_tc_tiling_on_sc=False)

    agg1 = pl.kernel(
        functools.partial(_agg1_body, chunks_per_tile=chunks_per_tile,
                          npad=npad),
        out_type=[jax.ShapeDtypeStruct((_NC, npad, d_hid), _F32),
                  jax.ShapeDtypeStruct((_NC, 2 * npad, 16), _F32)],
        mesh=mesh,
        scratch_types=[
            pltpu.VMEM_SHARED((npad, d_hid), _F32),
            pltpu.VMEM_SHARED((2 * npad, 16), _F32),
            pltpu.VMEM((2, _CH), _I32),
            pltpu.VMEM((_CH,), _I32),
            pltpu.VMEM((_CH, d_hid), _F32),
            pltpu.VMEM((_CH, 16), _F32),
        ],
        compiler_params=sc_params,
    )
    s1, l16 = agg1(y1p, sd, z32, z16, ones_h)

    nboff = npad // _ROWBLK
    h, stats = pl.pallas_call(
        _mid_body,
        grid=(nblk,),
        in_specs=[pl.BlockSpec((_NC, _ROWBLK, d_hid), lambda i: (0, i, 0)),
                  pl.BlockSpec((_NC, _ROWBLK, 16), lambda i: (0, i, 0)),
                  pl.BlockSpec((_NC, _ROWBLK, 16),
                               lambda i: (0, i + nboff, 0)),
                  pl.BlockSpec((_ROWBLK, d_hid), lambda i: (i, 0)),
                  pl.BlockSpec((1, d_hid), lambda i: (0, 0))],
        out_specs=[pl.BlockSpec((_ROWBLK, d_hid), lambda i: (i, 0)),
                   pl.BlockSpec((_ROWBLK, 16), lambda i: (i, 0))],
        out_shape=[jax.ShapeDtypeStruct((npad, d_hid), _F32),
                   jax.ShapeDtypeStruct((npad, 16), _F32)],
    )(s1, l16, l16, y1p, b1.reshape(1, d_hid))

    agg2 = pl.kernel(
        functools.partial(_agg2_body, chunks_per_tile=chunks_per_tile,
                          npad=npad),
        out_type=jax.ShapeDtypeStruct((_NC, npad, d_hid), _F32),
        mesh=mesh,
        scratch_types=[
            pltpu.VMEM_SHARED((npad, d_hid), _F32),
            pltpu.VMEM((2, _CH), _I32),
            pltpu.VMEM((_CH, d_hid), _F32),
        ],
        compiler_params=sc_params,
    )
    s2 = agg2(h, sd, z32)

    out = pl.pallas_call(
        _fin_body,
        grid=(nblk,),
        in_specs=[pl.BlockSpec((_NC, _ROWBLK, d_hid), lambda i: (0, i, 0)),
                  pl.BlockSpec((_ROWBLK, d_hid), lambda i: (i, 0)),
                  pl.BlockSpec((_ROWBLK, 16), lambda i: (i, 0)),
                  pl.BlockSpec((d_hid, d_out), lambda i: (0, 0)),
                  pl.BlockSpec((1, d_out), lambda i: (0, 0))],
        out_specs=pl.BlockSpec((_ROWBLK, d_out), lambda i: (i, 0)),
        out_shape=jax.ShapeDtypeStruct((npad, d_out), _F32),
    )(s2, h, stats, W2, b2.reshape(1, d_out))

    return out[:n]


# speedup vs baseline: 10.4564x; 1.0841x over previous
import functools

import jax
import jax.numpy as jnp
from jax import lax
from jax.experimental import pallas as pl
from jax.experimental.pallas import tpu as pltpu
from jax.experimental.pallas import tpu_sc as plsc

_F32 = jnp.float32
_I32 = jnp.int32

_NC = 2
_NS = 16
_NW = _NC * _NS
_CH = 512
_CHR = _CH // 128
_ROWBLK = 512


def _agg1_body(y1, sd, z32, z16, ones_h, s1, l16,
               acc32, acc16, sd_v, midx_v, rows_v, ones_v,
               sa0, sa1, sb0, sb1, sc0, sc1, sd0, sd1,
               *, chunks_per_tile, npad):
    c = lax.axis_index("c")
    s = lax.axis_index("s")
    wid = c * _NS + s
    rpt32 = npad // _NS
    rpt16 = 2 * npad // _NS
    sa = (sa0, sa1)
    sb = (sb0, sb1)
    sc = (sc0, sc1)
    sdm = (sd0, sd1)
    pltpu.sync_copy(z32.at[pl.ds(0, rpt32)], acc32.at[pl.ds(s * rpt32, rpt32)])
    pltpu.sync_copy(z16, acc16.at[pl.ds(s * rpt16, rpt16)])
    pltpu.sync_copy(ones_h, ones_v)
    plsc.subcore_barrier()
    base = wid * chunks_per_tile

    @pl.loop(0, chunks_per_tile // 2)
    def _pair(p):
        for b in range(2):
            i = p * 2 + b
            row = base + i

            @pl.when(p > 0)
            def _():
                pltpu.make_async_copy(
                    rows_v.at[b], acc32.at[sd_v.at[b, 1]], sc[b]).wait()
                pltpu.make_async_copy(
                    ones_v, acc16.at[midx_v.at[b]], sdm[b]).wait()

            pltpu.async_copy(sd.at[row], sd_v.at[b], sa[b])
            pltpu.make_async_copy(sd.at[row], sd_v.at[b], sa[b]).wait()
            pltpu.async_copy(y1.at[sd_v.at[b, 0]], rows_v.at[b], sb[b])
            for j in range(_CH // 16):
                sl = pl.ds(j * 16, 16)
                sv = sd_v[b, 0, sl]
                dv = sd_v[b, 1, sl]
                midx_v[b, sl] = dv + jnp.where(sv == dv, npad, 0).astype(_I32)
            pltpu.make_async_copy(
                y1.at[sd_v.at[b, 0]], rows_v.at[b], sb[b]).wait()
            pltpu.async_copy(rows_v.at[b], acc32.at[sd_v.at[b, 1]], sc[b],
                             add=True)
            pltpu.async_copy(ones_v, acc16.at[midx_v.at[b]], sdm[b],
                             add=True)

    for b in range(2):
        pltpu.make_async_copy(
            rows_v.at[b], acc32.at[sd_v.at[b, 1]], sc[b]).wait()
        pltpu.make_async_copy(ones_v, acc16.at[midx_v.at[b]], sdm[b]).wait()

    plsc.subcore_barrier()
    sl32 = pl.ds(s * rpt32, rpt32)
    sl16 = pl.ds(s * rpt16, rpt16)
    pltpu.sync_copy(acc32.at[sl32], s1.at[c, sl32])
    pltpu.sync_copy(acc16.at[sl16], l16.at[c, sl16])


def _agg2_body(h, sd, z32, s2,
               acc32, sd_v, rows_v,
               sa0, sa1, sb0, sb1, sc0, sc1,
               *, chunks_per_tile, npad):
    c = lax.axis_index("c")
    s = lax.axis_index("s")
    wid = c * _NS + s
    rpt32 = npad // _NS
    sa = (sa0, sa1)
    sb = (sb0, sb1)
    sc = (sc0, sc1)
    pltpu.sync_copy(z32.at[pl.ds(0, rpt32)], acc32.at[pl.ds(s * rpt32, rpt32)])
    plsc.subcore_barrier()
    base = wid * chunks_per_tile

    @pl.loop(0, chunks_per_tile // 2)
    def _pair(p):
        for b in range(2):
            i = p * 2 + b
            row = base + i

            @pl.when(p > 0)
            def _():
                pltpu.make_async_copy(
                    rows_v.at[b], acc32.at[sd_v.at[b, 1]], sc[b]).wait()

            pltpu.async_copy(sd.at[row], sd_v.at[b], sa[b])
            pltpu.make_async_copy(sd.at[row], sd_v.at[b], sa[b]).wait()
            pltpu.async_copy(h.at[sd_v.at[b, 0]], rows_v.at[b], sb[b])
            pltpu.make_async_copy(
                h.at[sd_v.at[b, 0]], rows_v.at[b], sb[b]).wait()
            pltpu.async_copy(rows_v.at[b], acc32.at[sd_v.at[b, 1]], sc[b],
                             add=True)

    for b in range(2):
        pltpu.make_async_copy(
            rows_v.at[b], acc32.at[sd_v.at[b, 1]], sc[b]).wait()

    plsc.subcore_barrier()
    sl32 = pl.ds(s * rpt32, rpt32)
    pltpu.sync_copy(acc32.at[sl32], s2.at[c, sl32])


def _mm1_body(x_ref, w_ref, o_ref):
    o_ref[...] = jnp.dot(x_ref[...], w_ref[...],
                         preferred_element_type=_F32)


def _mid_body(s1_ref, la_ref, lb_ref, y1_ref, b1_ref, h_ref, st_ref):
    seg = s1_ref[0] + s1_ref[1]
    la = la_ref[0] + la_ref[1]
    lb = lb_ref[0] + lb_ref[1]
    loop_cnt = lb[:, 0:1]
    deg_raw = la[:, 0:1] + loop_cnt
    missing = jnp.where(loop_cnt > 0.0, 0.0, 1.0)
    deg = jnp.maximum(deg_raw + missing, 1.0)
    y1 = y1_ref[...]
    h = jnp.maximum((seg + y1 * missing) / deg + b1_ref[...], 0.0)
    h_ref[...] = h
    nrows = missing.shape[0]
    st_ref[...] = jnp.concatenate(
        [missing, deg, jnp.zeros((nrows, 14), _F32)], axis=1)


def _fin_body(s2_ref, h_ref, st_ref, w2_ref, b2_ref, o_ref):
    seg = s2_ref[0] + s2_ref[1]
    missing = st_ref[:, 0:1]
    deg = st_ref[:, 1:2]
    aggr = (seg + h_ref[...] * missing) / deg
    z = jnp.dot(aggr, w2_ref[...], preferred_element_type=_F32) + b2_ref[...]
    m = jnp.max(z, axis=1, keepdims=True)
    e = jnp.exp(z - m)
    lse = jnp.log(jnp.sum(e, axis=1, keepdims=True))
    o_ref[...] = z - m - lse


def kernel(x, edge_index, W1, b1, W2, b2):
    n, d_in = x.shape
    e = edge_index.shape[1]
    d_hid = W1.shape[1]
    d_out = W2.shape[1]

    npad = ((n + 2 * _ROWBLK - 1) // _ROWBLK) * _ROWBLK
    trash = npad - 1
    chunks_per_tile = 2 * -(-e // (2 * _CH * _NW))
    epad = chunks_per_tile * _NW * _CH
    nblk = npad // _ROWBLK

    src = edge_index[0]
    dst = edge_index[1]
    pad = jnp.full((epad - e,), trash, _I32)
    srcl = jnp.concatenate([src, pad]).reshape(-1, _CH)
    dstl = jnp.concatenate([dst, pad]).reshape(-1, _CH)
    sd = jnp.stack([srcl, dstl], axis=1)
    x_pad = jnp.pad(x, ((0, npad - n), (0, 0)))
    z32 = jnp.zeros((npad // _NS + 8, d_hid), _F32)
    z16 = jnp.zeros((2 * npad // _NS, 16), _F32)
    ones_h = jnp.zeros((_CH, 16), _F32).at[:, 0].set(1.0)

    y1p = pl.pallas_call(
        _mm1_body,
        grid=(nblk,),
        in_specs=[pl.BlockSpec((_ROWBLK, d_in), lambda i: (i, 0)),
                  pl.BlockSpec((d_in, d_hid), lambda i: (0, 0))],
        out_specs=pl.BlockSpec((_ROWBLK, d_hid), lambda i: (i, 0)),
        out_shape=jax.ShapeDtypeStruct((npad, d_hid), _F32),
    )(x_pad, W1)

    mesh = plsc.VectorSubcoreMesh(core_axis_name="c", subcore_axis_name="s",
                                  num_cores=_NC, num_subcores=_NS)
    sc_params = pltpu.CompilerParams(use_tc_tiling_on_sc=False)

    agg1 = pl.kernel(
        functools.partial(_agg1_body, chunks_per_tile=chunks_per_tile,
                          npad=npad),
        out_type=[jax.ShapeDtypeStruct((_NC, npad, d_hid), _F32),
                  jax.ShapeDtypeStruct((_NC, 2 * npad, 16), _F32)],
        mesh=mesh,
        scratch_types=[
            pltpu.VMEM_SHARED((npad, d_hid), _F32),
            pltpu.VMEM_SHARED((2 * npad, 16), _F32),
            pltpu.VMEM((2, 2, _CH), _I32),
            pltpu.VMEM((2, _CH), _I32),
            pltpu.VMEM((2, _CH, d_hid), _F32),
            pltpu.VMEM((_CH, 16), _F32),
        ] + [pltpu.SemaphoreType.DMA] * 8,
        compiler_params=sc_params,
    )
    s1, l16 = agg1(y1p, sd, z32, z16, ones_h)

    nboff = npad // _ROWBLK
    h, stats = pl.pallas_call(
        _mid_body,
        grid=(nblk,),
        in_specs=[pl.BlockSpec((_NC, _ROWBLK, d_hid), lambda i: (0, i, 0)),
                  pl.BlockSpec((_NC, _ROWBLK, 16), lambda i: (0, i, 0)),
                  pl.BlockSpec((_NC, _ROWBLK, 16),
                               lambda i: (0, i + nboff, 0)),
                  pl.BlockSpec((_ROWBLK, d_hid), lambda i: (i, 0)),
                  pl.BlockSpec((1, d_hid), lambda i: (0, 0))],
        out_specs=[pl.BlockSpec((_ROWBLK, d_hid), lambda i: (i, 0)),
                   pl.BlockSpec((_ROWBLK, 16), lambda i: (i, 0))],
        out_shape=[jax.ShapeDtypeStruct((npad, d_hid), _F32),
                   jax.ShapeDtypeStruct((npad, 16), _F32)],
    )(s1, l16, l16, y1p, b1.reshape(1, d_hid))

    agg2 = pl.kernel(
        functools.partial(_agg2_body, chunks_per_tile=chunks_per_tile,
                          npad=npad),
        out_type=jax.ShapeDtypeStruct((_NC, npad, d_hid), _F32),
        mesh=mesh,
        scratch_types=[
            pltpu.VMEM_SHARED((npad, d_hid), _F32),
            pltpu.VMEM((2, 2, _CH), _I32),
            pltpu.VMEM((2, _CH, d_hid), _F32),
        ] + [pltpu.SemaphoreType.DMA] * 6,
        compiler_params=sc_params,
    )
    s2 = agg2(h, sd, z32)

    out = pl.pallas_call(
        _fin_body,
        grid=(nblk,),
        in_specs=[pl.BlockSpec((_NC, _ROWBLK, d_hid), lambda i: (0, i, 0)),
                  pl.BlockSpec((_ROWBLK, d_hid), lambda i: (i, 0)),
                  pl.BlockSpec((_ROWBLK, 16), lambda i: (i, 0)),
                  pl.BlockSpec((d_hid, d_out), lambda i: (0, 0)),
                  pl.BlockSpec((1, d_out), lambda i: (0, 0))],
        out_specs=pl.BlockSpec((_ROWBLK, d_out), lambda i: (i, 0)),
        out_shape=jax.ShapeDtypeStruct((npad, d_out), _F32),
    )(s2, h, stats, W2, b2.reshape(1, d_out))

    return out[:n]


# speedup vs baseline: 11.2576x; 1.0766x over previous
import functools

import jax
import jax.numpy as jnp
from jax import lax
from jax.experimental import pallas as pl
from jax.experimental.pallas import tpu as pltpu
from jax.experimental.pallas import tpu_sc as plsc

_F32 = jnp.float32
_I32 = jnp.int32

_NC = 2
_NS = 16
_NW = _NC * _NS
_CH = 512
_CHR = _CH // 128
_ROWBLK = 512


def _agg1_body(y1, sd, z32, z16, ones_h, s1, l16,
               acc32, acc16, sd_v, midx_v, rows_v, ones_v,
               sa0, sa1, sb0, sb1, sc0, sc1, sd0, sd1,
               *, cpt0, cpt1, npad):
    c = lax.axis_index("c")
    s = lax.axis_index("s")
    rpt32 = npad // _NS
    rpt16 = 2 * npad // _NS
    sa = (sa0, sa1)
    sb = (sb0, sb1)
    sc = (sc0, sc1)
    sdm = (sd0, sd1)
    pltpu.sync_copy(z32.at[pl.ds(0, rpt32)], acc32.at[pl.ds(s * rpt32, rpt32)])
    pltpu.sync_copy(z16, acc16.at[pl.ds(s * rpt16, rpt16)])
    pltpu.sync_copy(ones_h, ones_v)
    plsc.subcore_barrier()
    base = jnp.where(c == 0, s * cpt0, _NS * cpt0 + s * cpt1)
    npairs = jnp.where(c == 0, cpt0 // 2, cpt1 // 2)

    @pl.loop(0, npairs)
    def _pair(p):
        for b in range(2):
            i = p * 2 + b
            row = base + i

            @pl.when(p > 0)
            def _():
                pltpu.make_async_copy(
                    rows_v.at[b], acc32.at[sd_v.at[b, 1]], sc[b]).wait()
                pltpu.make_async_copy(
                    ones_v, acc16.at[midx_v.at[b]], sdm[b]).wait()

            pltpu.async_copy(sd.at[row], sd_v.at[b], sa[b])
            pltpu.make_async_copy(sd.at[row], sd_v.at[b], sa[b]).wait()
            pltpu.async_copy(y1.at[sd_v.at[b, 0]], rows_v.at[b], sb[b])
            for j in range(_CH // 16):
                sl = pl.ds(j * 16, 16)
                sv = sd_v[b, 0, sl]
                dv = sd_v[b, 1, sl]
                midx_v[b, sl] = dv + jnp.where(sv == dv, npad, 0).astype(_I32)
            pltpu.make_async_copy(
                y1.at[sd_v.at[b, 0]], rows_v.at[b], sb[b]).wait()
            pltpu.async_copy(rows_v.at[b], acc32.at[sd_v.at[b, 1]], sc[b],
                             add=True)
            pltpu.async_copy(ones_v, acc16.at[midx_v.at[b]], sdm[b],
                             add=True)

    for b in range(2):
        pltpu.make_async_copy(
            rows_v.at[b], acc32.at[sd_v.at[b, 1]], sc[b]).wait()
        pltpu.make_async_copy(ones_v, acc16.at[midx_v.at[b]], sdm[b]).wait()

    plsc.subcore_barrier()
    sl32 = pl.ds(s * rpt32, rpt32)
    sl16 = pl.ds(s * rpt16, rpt16)
    pltpu.sync_copy(acc32.at[sl32], s1.at[c, sl32])
    pltpu.sync_copy(acc16.at[sl16], l16.at[c, sl16])


def _agg2_body(h, sd, z32, s2,
               acc32, sd_v, rows_v,
               sa0, sa1, sb0, sb1, sc0, sc1,
               *, cpt0, cpt1, npad):
    c = lax.axis_index("c")
    s = lax.axis_index("s")
    wid = c * _NS + s
    rpt32 = npad // _NS
    sa = (sa0, sa1)
    sb = (sb0, sb1)
    sc = (sc0, sc1)
    pltpu.sync_copy(z32.at[pl.ds(0, rpt32)], acc32.at[pl.ds(s * rpt32, rpt32)])
    plsc.subcore_barrier()
    base = jnp.where(c == 0, s * cpt0, _NS * cpt0 + s * cpt1)
    npairs = jnp.where(c == 0, cpt0 // 2, cpt1 // 2)

    @pl.loop(0, npairs)
    def _pair(p):
        for b in range(2):
            i = p * 2 + b
            row = base + i

            @pl.when(p > 0)
            def _():
                pltpu.make_async_copy(
                    rows_v.at[b], acc32.at[sd_v.at[b, 1]], sc[b]).wait()

            pltpu.async_copy(sd.at[row], sd_v.at[b], sa[b])
            pltpu.make_async_copy(sd.at[row], sd_v.at[b], sa[b]).wait()
            pltpu.async_copy(h.at[sd_v.at[b, 0]], rows_v.at[b], sb[b])
            pltpu.make_async_copy(
                h.at[sd_v.at[b, 0]], rows_v.at[b], sb[b]).wait()
            pltpu.async_copy(rows_v.at[b], acc32.at[sd_v.at[b, 1]], sc[b],
                             add=True)

    for b in range(2):
        pltpu.make_async_copy(
            rows_v.at[b], acc32.at[sd_v.at[b, 1]], sc[b]).wait()

    plsc.subcore_barrier()
    sl32 = pl.ds(s * rpt32, rpt32)
    pltpu.sync_copy(acc32.at[sl32], s2.at[c, sl32])


def _mm1_body(x_ref, w_ref, o_ref):
    o_ref[...] = jnp.dot(x_ref[...], w_ref[...],
                         preferred_element_type=_F32)


def _mid_body(s1_ref, la_ref, lb_ref, y1_ref, b1_ref, h_ref, st_ref):
    seg = s1_ref[0] + s1_ref[1]
    la = la_ref[0] + la_ref[1]
    lb = lb_ref[0] + lb_ref[1]
    loop_cnt = lb[:, 0:1]
    deg_raw = la[:, 0:1] + loop_cnt
    missing = jnp.where(loop_cnt > 0.0, 0.0, 1.0)
    deg = jnp.maximum(deg_raw + missing, 1.0)
    y1 = y1_ref[...]
    h = jnp.maximum((seg + y1 * missing) / deg + b1_ref[...], 0.0)
    h_ref[...] = h
    nrows = missing.shape[0]
    st_ref[...] = jnp.concatenate(
        [missing, deg, jnp.zeros((nrows, 14), _F32)], axis=1)


def _fin_body(s2_ref, h_ref, st_ref, w2_ref, b2_ref, o_ref):
    seg = s2_ref[0] + s2_ref[1]
    missing = st_ref[:, 0:1]
    deg = st_ref[:, 1:2]
    aggr = (seg + h_ref[...] * missing) / deg
    z = jnp.dot(aggr, w2_ref[...], preferred_element_type=_F32) + b2_ref[...]
    m = jnp.max(z, axis=1, keepdims=True)
    e = jnp.exp(z - m)
    lse = jnp.log(jnp.sum(e, axis=1, keepdims=True))
    o_ref[...] = z - m - lse


def kernel(x, edge_index, W1, b1, W2, b2):
    n, d_in = x.shape
    e = edge_index.shape[1]
    d_hid = W1.shape[1]
    d_out = W2.shape[1]

    npad = ((n + 2 * _ROWBLK - 1) // _ROWBLK) * _ROWBLK
    trash = npad - 1
    heavy = int(round(e * 0.666))
    cpt0 = 2 * -(-heavy // (2 * _CH * _NS))
    cpt1 = 2 * -(-(e - _NS * cpt0 * _CH) // (2 * _CH * _NS))
    epad = (cpt0 + cpt1) * _NS * _CH
    nblk = npad // _ROWBLK

    src = edge_index[0]
    dst = edge_index[1]
    pad = jnp.full((epad - e,), trash, _I32)
    srcl = jnp.concatenate([src, pad]).reshape(-1, _CH)
    dstl = jnp.concatenate([dst, pad]).reshape(-1, _CH)
    sd = jnp.stack([srcl, dstl], axis=1)
    x_pad = jnp.pad(x, ((0, npad - n), (0, 0)))
    z32 = jnp.zeros((npad // _NS + 8, d_hid), _F32)
    z16 = jnp.zeros((2 * npad // _NS, 16), _F32)
    ones_h = jnp.zeros((_CH, 16), _F32).at[:, 0].set(1.0)

    y1p = pl.pallas_call(
        _mm1_body,
        grid=(nblk,),
        in_specs=[pl.BlockSpec((_ROWBLK, d_in), lambda i: (i, 0)),
                  pl.BlockSpec((d_in, d_hid), lambda i: (0, 0))],
        out_specs=pl.BlockSpec((_ROWBLK, d_hid), lambda i: (i, 0)),
        out_shape=jax.ShapeDtypeStruct((npad, d_hid), _F32),
    )(x_pad, W1)

    mesh = plsc.VectorSubcoreMesh(core_axis_name="c", subcore_axis_name="s",
                                  num_cores=_NC, num_subcores=_NS)
    sc_params = pltpu.CompilerParams(use_tc_tiling_on_sc=False)

    agg1 = pl.kernel(
        functools.partial(_agg1_body, cpt0=cpt0, cpt1=cpt1, npad=npad),
        out_type=[jax.ShapeDtypeStruct((_NC, npad, d_hid), _F32),
                  jax.ShapeDtypeStruct((_NC, 2 * npad, 16), _F32)],
        mesh=mesh,
        scratch_types=[
            pltpu.VMEM_SHARED((npad, d_hid), _F32),
            pltpu.VMEM_SHARED((2 * npad, 16), _F32),
            pltpu.VMEM((2, 2, _CH), _I32),
            pltpu.VMEM((2, _CH), _I32),
            pltpu.VMEM((2, _CH, d_hid), _F32),
            pltpu.VMEM((_CH, 16), _F32),
        ] + [pltpu.SemaphoreType.DMA] * 8,
        compiler_params=sc_params,
    )
    s1, l16 = agg1(y1p, sd, z32, z16, ones_h)

    nboff = npad // _ROWBLK
    h, stats = pl.pallas_call(
        _mid_body,
        grid=(nblk,),
        in_specs=[pl.BlockSpec((_NC, _ROWBLK, d_hid), lambda i: (0, i, 0)),
                  pl.BlockSpec((_NC, _ROWBLK, 16), lambda i: (0, i, 0)),
                  pl.BlockSpec((_NC, _ROWBLK, 16),
                               lambda i: (0, i + nboff, 0)),
                  pl.BlockSpec((_ROWBLK, d_hid), lambda i: (i, 0)),
                  pl.BlockSpec((1, d_hid), lambda i: (0, 0))],
        out_specs=[pl.BlockSpec((_ROWBLK, d_hid), lambda i: (i, 0)),
                   pl.BlockSpec((_ROWBLK, 16), lambda i: (i, 0))],
        out_shape=[jax.ShapeDtypeStruct((npad, d_hid), _F32),
                   jax.ShapeDtypeStruct((npad, 16), _F32)],
    )(s1, l16, l16, y1p, b1.reshape(1, d_hid))

    agg2 = pl.kernel(
        functools.partial(_agg2_body, cpt0=cpt0, cpt1=cpt1, npad=npad),
        out_type=jax.ShapeDtypeStruct((_NC, npad, d_hid), _F32),
        mesh=mesh,
        scratch_types=[
            pltpu.VMEM_SHARED((npad, d_hid), _F32),
            pltpu.VMEM((2, 2, _CH), _I32),
            pltpu.VMEM((2, _CH, d_hid), _F32),
        ] + [pltpu.SemaphoreType.DMA] * 6,
        compiler_params=sc_params,
    )
    s2 = agg2(h, sd, z32)

    out = pl.pallas_call(
        _fin_body,
        grid=(nblk,),
        in_specs=[pl.BlockSpec((_NC, _ROWBLK, d_hid), lambda i: (0, i, 0)),
                  pl.BlockSpec((_ROWBLK, d_hid), lambda i: (i, 0)),
                  pl.BlockSpec((_ROWBLK, 16), lambda i: (i, 0)),
                  pl.BlockSpec((d_hid, d_out), lambda i: (0, 0)),
                  pl.BlockSpec((1, d_out), lambda i: (0, 0))],
        out_specs=pl.BlockSpec((_ROWBLK, d_out), lambda i: (i, 0)),
        out_shape=jax.ShapeDtypeStruct((npad, d_out), _F32),
    )(s2, h, stats, W2, b2.reshape(1, d_out))

    return out[:n]


# speedup vs baseline: 11.7234x; 1.0414x over previous
import functools

import jax
import jax.numpy as jnp
from jax import lax
from jax.experimental import pallas as pl
from jax.experimental.pallas import tpu as pltpu
from jax.experimental.pallas import tpu_sc as plsc

_F32 = jnp.float32
_I32 = jnp.int32

_NC = 2
_NS = 16
_NW = _NC * _NS
_CH = 512
_CHR = _CH // 128
_ROWBLK = 512


def _agg1_body(y1, sd, z32, z16, ones_h, s1, l16,
               acc32, acc16, sd_v, midx_v, rows_v, ones_v,
               sa0, sa1, sb0, sb1, sc0, sc1, sd0, sd1,
               *, cpt0, cpt1, npad):
    c = lax.axis_index("c")
    s = lax.axis_index("s")
    rpt32 = npad // _NS
    rpt16 = 2 * npad // _NS
    sa = (sa0, sa1)
    sb = (sb0, sb1)
    sc = (sc0, sc1)
    sdm = (sd0, sd1)
    pltpu.sync_copy(z32.at[pl.ds(0, rpt32)], acc32.at[pl.ds(s * rpt32, rpt32)])
    pltpu.sync_copy(z16, acc16.at[pl.ds(s * rpt16, rpt16)])
    pltpu.sync_copy(ones_h, ones_v)
    plsc.subcore_barrier()
    base = jnp.where(c == 0, s * cpt0, _NS * cpt0 + s * cpt1)
    npairs = jnp.where(c == 0, cpt0 // 2, cpt1 // 2)

    @pl.loop(0, npairs)
    def _pair(p):
        for b in range(2):
            i = p * 2 + b
            row = base + i

            @pl.when(p > 0)
            def _():
                pltpu.make_async_copy(
                    rows_v.at[b], acc32.at[sd_v.at[b, 1]], sc[b]).wait()
                pltpu.make_async_copy(
                    ones_v, acc16.at[midx_v.at[b]], sdm[b]).wait()

            pltpu.async_copy(sd.at[row], sd_v.at[b], sa[b])
            pltpu.make_async_copy(sd.at[row], sd_v.at[b], sa[b]).wait()
            pltpu.async_copy(y1.at[sd_v.at[b, 0]], rows_v.at[b], sb[b])
            for j in range(_CH // 16):
                sl = pl.ds(j * 16, 16)
                sv = sd_v[b, 0, sl]
                dv = sd_v[b, 1, sl]
                midx_v[b, sl] = dv + jnp.where(sv == dv, npad, 0).astype(_I32)
            pltpu.make_async_copy(
                y1.at[sd_v.at[b, 0]], rows_v.at[b], sb[b]).wait()
            pltpu.async_copy(rows_v.at[b], acc32.at[sd_v.at[b, 1]], sc[b],
                             add=True)
            pltpu.async_copy(ones_v, acc16.at[midx_v.at[b]], sdm[b],
                             add=True)

    for b in range(2):
        pltpu.make_async_copy(
            rows_v.at[b], acc32.at[sd_v.at[b, 1]], sc[b]).wait()
        pltpu.make_async_copy(ones_v, acc16.at[midx_v.at[b]], sdm[b]).wait()

    plsc.subcore_barrier()
    sl32 = pl.ds(s * rpt32, rpt32)
    sl16 = pl.ds(s * rpt16, rpt16)
    pltpu.sync_copy(acc32.at[sl32], s1.at[c, sl32])
    pltpu.sync_copy(acc16.at[sl16], l16.at[c, sl16])


def _agg2_body(h, sd, z32, s2,
               acc32, sd_v, rows_v,
               sa0, sa1, sb0, sb1, sc0, sc1,
               *, cpt0, cpt1, npad):
    c = lax.axis_index("c")
    s = lax.axis_index("s")
    wid = c * _NS + s
    rpt32 = npad // _NS
    sa = (sa0, sa1)
    sb = (sb0, sb1)
    sc = (sc0, sc1)
    pltpu.sync_copy(z32.at[pl.ds(0, rpt32)], acc32.at[pl.ds(s * rpt32, rpt32)])
    plsc.subcore_barrier()
    base = jnp.where(c == 0, s * cpt0, _NS * cpt0 + s * cpt1)
    npairs = jnp.where(c == 0, cpt0 // 2, cpt1 // 2)

    @pl.loop(0, npairs)
    def _pair(p):
        for b in range(2):
            i = p * 2 + b
            row = base + i

            @pl.when(p > 0)
            def _():
                pltpu.make_async_copy(
                    rows_v.at[b], acc32.at[sd_v.at[b, 1]], sc[b]).wait()

            pltpu.async_copy(sd.at[row], sd_v.at[b], sa[b])
            pltpu.make_async_copy(sd.at[row], sd_v.at[b], sa[b]).wait()
            pltpu.async_copy(h.at[sd_v.at[b, 0]], rows_v.at[b], sb[b])
            pltpu.make_async_copy(
                h.at[sd_v.at[b, 0]], rows_v.at[b], sb[b]).wait()
            pltpu.async_copy(rows_v.at[b], acc32.at[sd_v.at[b, 1]], sc[b],
                             add=True)

    for b in range(2):
        pltpu.make_async_copy(
            rows_v.at[b], acc32.at[sd_v.at[b, 1]], sc[b]).wait()

    plsc.subcore_barrier()
    sl32 = pl.ds(s * rpt32, rpt32)
    pltpu.sync_copy(acc32.at[sl32], s2.at[c, sl32])


def _mm1_body(x_ref, w_ref, o_ref):
    o_ref[...] = jnp.dot(x_ref[...], w_ref[...],
                         preferred_element_type=_F32)


def _mid_body(s1_ref, la_ref, lb_ref, y1_ref, b1_ref, h_ref, st_ref):
    seg = s1_ref[0] + s1_ref[1]
    la = la_ref[0] + la_ref[1]
    lb = lb_ref[0] + lb_ref[1]
    loop_cnt = lb[:, 0:1]
    deg_raw = la[:, 0:1] + loop_cnt
    missing = jnp.where(loop_cnt > 0.0, 0.0, 1.0)
    deg = jnp.maximum(deg_raw + missing, 1.0)
    y1 = y1_ref[...]
    h = jnp.maximum((seg + y1 * missing) / deg + b1_ref[...], 0.0)
    h_ref[...] = h
    nrows = missing.shape[0]
    st_ref[...] = jnp.concatenate(
        [missing, deg, jnp.zeros((nrows, 14), _F32)], axis=1)


def _fin_body(s2_ref, h_ref, st_ref, w2_ref, b2_ref, o_ref):
    seg = s2_ref[0] + s2_ref[1]
    missing = st_ref[:, 0:1]
    deg = st_ref[:, 1:2]
    aggr = (seg + h_ref[...] * missing) / deg
    z = jnp.dot(aggr, w2_ref[...], preferred_element_type=_F32) + b2_ref[...]
    m = jnp.max(z, axis=1, keepdims=True)
    e = jnp.exp(z - m)
    lse = jnp.log(jnp.sum(e, axis=1, keepdims=True))
    o_ref[...] = z - m - lse


def kernel(x, edge_index, W1, b1, W2, b2):
    n, d_in = x.shape
    e = edge_index.shape[1]
    d_hid = W1.shape[1]
    d_out = W2.shape[1]

    npad = ((n + 2 * _ROWBLK - 1) // _ROWBLK) * _ROWBLK
    trash = npad - 1
    heavy = int(round(e * 0.84))
    cpt0 = 2 * -(-heavy // (2 * _CH * _NS))
    cpt1 = 2 * -(-(e - _NS * cpt0 * _CH) // (2 * _CH * _NS))
    epad = (cpt0 + cpt1) * _NS * _CH
    nblk = npad // _ROWBLK

    src = edge_index[0]
    dst = edge_index[1]
    pad = jnp.full((epad - e,), trash, _I32)
    srcl = jnp.concatenate([src, pad]).reshape(-1, _CH)
    dstl = jnp.concatenate([dst, pad]).reshape(-1, _CH)
    sd = jnp.stack([srcl, dstl], axis=1)
    x_pad = jnp.pad(x, ((0, npad - n), (0, 0)))
    z32 = jnp.zeros((npad // _NS + 8, d_hid), _F32)
    z16 = jnp.zeros((2 * npad // _NS, 16), _F32)
    ones_h = jnp.zeros((_CH, 16), _F32).at[:, 0].set(1.0)

    y1p = pl.pallas_call(
        _mm1_body,
        grid=(nblk,),
        in_specs=[pl.BlockSpec((_ROWBLK, d_in), lambda i: (i, 0)),
                  pl.BlockSpec((d_in, d_hid), lambda i: (0, 0))],
        out_specs=pl.BlockSpec((_ROWBLK, d_hid), lambda i: (i, 0)),
        out_shape=jax.ShapeDtypeStruct((npad, d_hid), _F32),
    )(x_pad, W1)

    mesh = plsc.VectorSubcoreMesh(core_axis_name="c", subcore_axis_name="s",
                                  num_cores=_NC, num_subcores=_NS)
    sc_params = pltpu.CompilerParams(use_tc_tiling_on_sc=False)

    agg1 = pl.kernel(
        functools.partial(_agg1_body, cpt0=cpt0, cpt1=cpt1, npad=npad),
        out_type=[jax.ShapeDtypeStruct((_NC, npad, d_hid), _F32),
                  jax.ShapeDtypeStruct((_NC, 2 * npad, 16), _F32)],
        mesh=mesh,
        scratch_types=[
            pltpu.VMEM_SHARED((npad, d_hid), _F32),
            pltpu.VMEM_SHARED((2 * npad, 16), _F32),
            pltpu.VMEM((2, 2, _CH), _I32),
            pltpu.VMEM((2, _CH), _I32),
            pltpu.VMEM((2, _CH, d_hid), _F32),
            pltpu.VMEM((_CH, 16), _F32),
        ] + [pltpu.SemaphoreType.DMA] * 8,
        compiler_params=sc_params,
    )
    s1, l16 = agg1(y1p, sd, z32, z16, ones_h)

    nboff = npad // _ROWBLK
    h, stats = pl.pallas_call(
        _mid_body,
        grid=(nblk,),
        in_specs=[pl.BlockSpec((_NC, _ROWBLK, d_hid), lambda i: (0, i, 0)),
                  pl.BlockSpec((_NC, _ROWBLK, 16), lambda i: (0, i, 0)),
                  pl.BlockSpec((_NC, _ROWBLK, 16),
                               lambda i: (0, i + nboff, 0)),
                  pl.BlockSpec((_ROWBLK, d_hid), lambda i: (i, 0)),
                  pl.BlockSpec((1, d_hid), lambda i: (0, 0))],
        out_specs=[pl.BlockSpec((_ROWBLK, d_hid), lambda i: (i, 0)),
                   pl.BlockSpec((_ROWBLK, 16), lambda i: (i, 0))],
        out_shape=[jax.ShapeDtypeStruct((npad, d_hid), _F32),
                   jax.ShapeDtypeStruct((npad, 16), _F32)],
    )(s1, l16, l16, y1p, b1.reshape(1, d_hid))

    agg2 = pl.kernel(
        functools.partial(_agg2_body, cpt0=cpt0, cpt1=cpt1, npad=npad),
        out_type=jax.ShapeDtypeStruct((_NC, npad, d_hid), _F32),
        mesh=mesh,
        scratch_types=[
            pltpu.VMEM_SHARED((npad, d_hid), _F32),
            pltpu.VMEM((2, 2, _CH), _I32),
            pltpu.VMEM((2, _CH, d_hid), _F32),
        ] + [pltpu.SemaphoreType.DMA] * 6,
        compiler_params=sc_params,
    )
    s2 = agg2(h, sd, z32)

    out = pl.pallas_call(
        _fin_body,
        grid=(nblk,),
        in_specs=[pl.BlockSpec((_NC, _ROWBLK, d_hid), lambda i: (0, i, 0)),
                  pl.BlockSpec((_ROWBLK, d_hid), lambda i: (i, 0)),
                  pl.BlockSpec((_ROWBLK, 16), lambda i: (i, 0)),
                  pl.BlockSpec((d_hid, d_out), lambda i: (0, 0)),
                  pl.BlockSpec((1, d_out), lambda i: (0, 0))],
        out_specs=pl.BlockSpec((_ROWBLK, d_out), lambda i: (i, 0)),
        out_shape=jax.ShapeDtypeStruct((npad, d_out), _F32),
    )(s2, h, stats, W2, b2.reshape(1, d_out))

    return out[:n]
